# baseline (speedup 1.0000x reference)
import jax
import jax.numpy as jnp
from jax import lax
from jax.experimental import pallas as pl
from jax.experimental.pallas import tpu as pltpu
from jax.experimental.pallas import tpu_sc as plsc

N = 10000
E = 320000
D = 128
EPS = 1e-5

NC = 2
NS = 16
L = 16
NW = NC * NS
EB = 128
NB = 79
NB3 = 168
CHUNK = 24
NCH = NB3 // CHUNK
EP = NW * NB * EB
EP2 = NS * NB3 * EB
NP = 10240
RPT = NP // NS
CD = D // NC
BNS = 1.0 / (1.0 + EPS) ** 0.5

_PERM64 = []
for _g in range(CD // 32):
    _PERM64 += [32 * _g + 2 * _k for _k in range(16)]
    _PERM64 += [32 * _g + 2 * _k + 1 for _k in range(16)]
_PERMFULL = _PERM64 + [CD + _p for _p in _PERM64]

_mesh = plsc.VectorSubcoreMesh(
    core_axis_name="c", subcore_axis_name="s", num_cores=NC, num_subcores=NS
)
_sc_params = pltpu.CompilerParams(
    needs_layout_passes=False, use_tc_tiling_on_sc=False
)


def _deg_body(srcp, dstp, degs_out, degd_out, sidx, didx, degs, degd):
    c = lax.axis_index("c")
    s = lax.axis_index("s")
    wid = c * NS + s
    pltpu.sync_copy(srcp.at[wid], sidx)
    pltpu.sync_copy(dstp.at[wid], didx)
    zeros = jnp.zeros((L,), jnp.float32)

    def zero_body(i, _):
        degs[pl.ds(i * L, L)] = zeros
        degd[pl.ds(i * L, L)] = zeros
        return 0

    lax.fori_loop(0, NP // L, zero_body, 0)
    ones = jnp.ones((L,), jnp.float32)

    def cnt(j, _):
        for g in range(EB // L):
            plsc.addupdate_scatter(degs, [sidx[j, pl.ds(g * L, L)]], ones)
            plsc.addupdate_scatter(degd, [didx[j, pl.ds(g * L, L)]], ones)
        return 0

    lax.fori_loop(0, NB, cnt, 0)
    pltpu.sync_copy(degs, degs_out.at[wid])
    pltpu.sync_copy(degd, degd_out.at[wid])


_deg_call = pl.kernel(
    _deg_body,
    out_type=(
        jax.ShapeDtypeStruct((NW, NP), jnp.float32),
        jax.ShapeDtypeStruct((NW, NP), jnp.float32),
    ),
    mesh=_mesh,
    scratch_types=[
        pltpu.VMEM((NB, EB), jnp.int32),
        pltpu.VMEM((NB, EB), jnp.int32),
        pltpu.VMEM((NP,), jnp.float32),
        pltpu.VMEM((NP,), jnp.float32),
    ],
    compiler_params=_sc_params,
)


def _msg_body(table, srcp, dstp, ewp, parts,
              sidxc, didxc, ewvc, b0, b1, b2, b3, f0, f1, f2, agg,
              gsem0, gsem1, gsem2, gsem3, ssem0, ssem1, ssem2):
    c = lax.axis_index("c")
    s = lax.axis_index("s")

    zeros = jnp.zeros((L,), jnp.float32)

    def zb(i, _):
        f0[i // (CD // L), pl.ds((i % (CD // L)) * L, L)] = zeros
        return 0

    lax.fori_loop(0, EB * (CD // L), zb, 0)

    def zs(k, _):
        pltpu.sync_copy(f0, agg.at[pl.ds(s * RPT + k * EB, EB)])
        return 0

    lax.fori_loop(0, RPT // EB, zs, 0)
    plsc.subcore_barrier()

    bbufs = (b0, b1, b2, b3)
    gsems = (gsem0, gsem1, gsem2, gsem3)
    fbufs = (f0, f1, f2)
    ssems = (ssem0, ssem1, ssem2)

    def gather(i, buf, sem):
        return pltpu.async_copy(table.at[c].at[sidxc.at[i]], buf, sem)

    def gather_wait(i, buf, sem):
        pltpu.make_async_copy(table.at[c].at[sidxc.at[i]], buf, sem).wait()

    def scatter(i, buf, sem):
        return pltpu.async_copy(buf, agg.at[didxc.at[i]], sem, add=True)

    def scatter_wait(i, buf, sem):
        pltpu.make_async_copy(buf, agg.at[didxc.at[i]], sem).wait()

    def scale(i, bsrc, fdst):
        def sc_body(g2, _):
            wv = ewvc[i, pl.ds(g2 * L, L)]
            for k in range(L):
                w = wv[k]
                e = g2 * L + k
                for g in range(CD // 32):
                    v = bsrc[e, pl.ds(g * 32, 32)]
                    a, b = plsc.unpack(v, format=plsc.PackFormat.INTERLEAVED)
                    fdst[e, pl.ds(g * 32, L)] = a * w
                    fdst[e, pl.ds(g * 32 + L, L)] = b * w
            return 0

        lax.fori_loop(0, EB // L, sc_body, 0)

    def chunk(k, _):
        @pl.when(k >= 1)
        def _():
            for t in range(3):
                scatter_wait(CHUNK - 3 + t, fbufs[t], ssems[t])

        base = k * CHUNK
        pltpu.sync_copy(srcp.at[s, pl.ds(base, CHUNK)], sidxc)
        pltpu.sync_copy(dstp.at[s, pl.ds(base, CHUNK)], didxc)
        pltpu.sync_copy(ewp.at[s, pl.ds(base, CHUNK)], ewvc)

        gather(0, b0, gsem0)
        gather(1, b1, gsem1)
        gather(2, b2, gsem2)
        for i in range(CHUNK):
            tb = i % 4
            tf = i % 3
            if i + 3 < CHUNK:
                gather(i + 3, bbufs[(i + 3) % 4], gsems[(i + 3) % 4])
            gather_wait(i, bbufs[tb], gsems[tb])
            if i >= 3:
                scatter_wait(i - 3, fbufs[tf], ssems[tf])
            scale(i, bbufs[tb], fbufs[tf])
            scatter(i, fbufs[tf], ssems[tf])
        return 0

    lax.fori_loop(0, NCH, chunk, 0)
    for t in range(3):
        scatter_wait(CHUNK - 3 + t, fbufs[t], ssems[t])
    plsc.subcore_barrier()
    pltpu.sync_copy(agg.at[pl.ds(s * RPT, RPT)], parts.at[c, pl.ds(s * RPT, RPT)])


_msg_call = pl.kernel(
    _msg_body,
    out_type=jax.ShapeDtypeStruct((NC, NP, CD), jnp.float32),
    mesh=_mesh,
    scratch_types=[
        pltpu.VMEM((CHUNK, EB), jnp.int32),
        pltpu.VMEM((CHUNK, EB), jnp.int32),
        pltpu.VMEM((CHUNK, EB), jnp.float32),
        pltpu.VMEM((EB, CD), jnp.bfloat16),
        pltpu.VMEM((EB, CD), jnp.bfloat16),
        pltpu.VMEM((EB, CD), jnp.bfloat16),
        pltpu.VMEM((EB, CD), jnp.bfloat16),
        pltpu.VMEM((EB, CD), jnp.float32),
        pltpu.VMEM((EB, CD), jnp.float32),
        pltpu.VMEM((EB, CD), jnp.float32),
        pltpu.VMEM_SHARED((NP, CD), jnp.float32),
        pltpu.SemaphoreType.DMA,
        pltpu.SemaphoreType.DMA,
        pltpu.SemaphoreType.DMA,
        pltpu.SemaphoreType.DMA,
        pltpu.SemaphoreType.DMA,
        pltpu.SemaphoreType.DMA,
        pltpu.SemaphoreType.DMA,
    ],
    compiler_params=_sc_params,
)


def _norm_body(ds_ref, dd_ref, ns_ref, nd_ref):
    degs = jnp.sum(ds_ref[...], axis=0)
    degd = jnp.sum(dd_ref[...], axis=0)
    ns_ref[...] = lax.rsqrt(jnp.maximum(degs, 1.0))
    nd_ref[...] = lax.rsqrt(jnp.maximum(degd, 1.0))


_norm_call = pl.pallas_call(
    _norm_body,
    out_shape=(
        jax.ShapeDtypeStruct((NP // 128, 128), jnp.float32),
        jax.ShapeDtypeStruct((NP // 128, 128), jnp.float32),
    ),
)


def _scalet_body(xs_ref, ns_ref, out_ref):
    out_ref[...] = (xs_ref[...] * ns_ref[...]).astype(jnp.bfloat16)


_scale_table = pl.pallas_call(
    _scalet_body,
    grid=(NP // 512,),
    in_specs=[
        pl.BlockSpec((NC, 512, CD), lambda i: (0, i, 0)),
        pl.BlockSpec((512, 1), lambda i: (i, 0)),
    ],
    out_specs=pl.BlockSpec((NC, 512, CD), lambda i: (0, i, 0)),
    out_shape=jax.ShapeDtypeStruct((NC, NP, CD), jnp.bfloat16),
)


def _make_dense(with_res, ns_scale):
    def body(*refs):
        refs = list(refs)
        parts_ref = refs.pop(0)
        nd_ref = refs.pop(0)
        ns_ref = refs.pop(0) if ns_scale else None
        w_ref, b_ref, g_ref, be_ref = refs[:4]
        refs = refs[4:]
        res_ref = refs.pop(0) if with_res else None
        out_ref = refs.pop(0)
        p = jnp.concatenate([parts_ref[0], parts_ref[1]], axis=1) * nd_ref[...]
        acc = jnp.dot(p, w_ref[...], preferred_element_type=jnp.float32)
        y = g_ref[...] * ((acc + b_ref[...]) * BNS) + be_ref[...]
        if with_res:
            y = y + jnp.concatenate([res_ref[0], res_ref[1]], axis=1)
        if ns_scale:
            y = y * ns_ref[...]
            out_ref[0] = y[:, :CD].astype(jnp.bfloat16)
            out_ref[1] = y[:, CD:].astype(jnp.bfloat16)
        else:
            out_ref[0] = y[:, :CD]
            out_ref[1] = y[:, CD:]

    R = 512
    in_specs = [
        pl.BlockSpec((NC, R, CD), lambda i: (0, i, 0)),
        pl.BlockSpec((R, 1), lambda i: (i, 0)),
    ]
    if ns_scale:
        in_specs.append(pl.BlockSpec((R, 1), lambda i: (i, 0)))
    in_specs += [
        pl.BlockSpec((D, D), lambda i: (0, 0)),
        pl.BlockSpec((1, D), lambda i: (0, 0)),
        pl.BlockSpec((1, D), lambda i: (0, 0)),
        pl.BlockSpec((1, D), lambda i: (0, 0)),
    ]
    if with_res:
        in_specs.append(pl.BlockSpec((NC, R, CD), lambda i: (0, i, 0)))
    out_dtype = jnp.bfloat16 if ns_scale else jnp.float32
    return pl.pallas_call(
        body,
        grid=(NP // R,),
        in_specs=in_specs,
        out_specs=pl.BlockSpec((NC, R, CD), lambda i: (0, i, 0)),
        out_shape=jax.ShapeDtypeStruct((NC, NP, CD), out_dtype),
    )


_dense_mid = _make_dense(False, True)
_dense_final = _make_dense(True, False)


def kernel(x, edge_index, edge_weight, W1, b1, g1, be1, W2, b2, g2, be2):
    src = edge_index[0]
    dst = edge_index[1]
    pad = EP - E
    pad2 = EP2 - E
    srcp32 = jnp.concatenate(
        [src, jnp.full((pad,), N, jnp.int32)]).reshape(NW, NB, EB)
    dstp32 = jnp.concatenate(
        [dst, jnp.full((pad,), N, jnp.int32)]).reshape(NW, NB, EB)
    srcp16 = jnp.concatenate(
        [src, jnp.full((pad2,), N, jnp.int32)]).reshape(NS, NB3, EB)
    dstp16 = jnp.concatenate(
        [dst, jnp.full((pad2,), N, jnp.int32)]).reshape(NS, NB3, EB)
    ewp16 = jnp.concatenate(
        [edge_weight, jnp.zeros((pad2,), jnp.float32)]).reshape(NS, NB3, EB)

    xp = jnp.zeros((NP, D), jnp.float32).at[:N].set(x)
    xs = jnp.stack([xp[:, :CD], xp[:, CD:]])

    degs_p, degd_p = _deg_call(srcp32, dstp32)
    ns80, nd80 = _norm_call(
        degs_p.reshape(NW, NP // 128, 128), degd_p.reshape(NW, NP // 128, 128)
    )
    nscol = ns80.reshape(NP, 1)
    ndcol = nd80.reshape(NP, 1)

    perm = jnp.array(_PERMFULL, dtype=jnp.int32)
    W1p = W1[perm, :]
    W2p = W2[perm, :]
    b1r, g1r, be1r = b1.reshape(1, D), g1.reshape(1, D), be1.reshape(1, D)
    b2r, g2r, be2r = b2.reshape(1, D), g2.reshape(1, D), be2.reshape(1, D)

    xss = _scale_table(xs, nscol)
    parts1 = _msg_call(xss, srcp16, dstp16, ewp16)
    h1s = _dense_mid(parts1, ndcol, nscol, W1p, b1r, g1r, be1r)
    parts2 = _msg_call(h1s, srcp16, dstp16, ewp16)
    out = _dense_final(parts2, ndcol, W2p, b2r, g2r, be2r, xs)
    return jnp.concatenate([out[0, :N], out[1, :N]], axis=1)


# speedup vs baseline: 1.2745x; 1.2745x over previous
import jax
import jax.numpy as jnp
from jax import lax
from jax.experimental import pallas as pl
from jax.experimental.pallas import tpu as pltpu
from jax.experimental.pallas import tpu_sc as plsc

N = 10000
E = 320000
D = 128
EPS = 1e-5

NC = 2
NS = 16
L = 16
NW = NC * NS
EB = 128
NB = 79
NB3 = 159
EP = NW * NB * EB
EP2 = NS * NB3 * EB
NP = 10240
RPT = NP // NS
CD = D // NC
BNS = 1.0 / (1.0 + EPS) ** 0.5

_PERM64 = []
for _g in range(CD // 32):
    _PERM64 += [32 * _g + 2 * _k for _k in range(16)]
    _PERM64 += [32 * _g + 2 * _k + 1 for _k in range(16)]
_PERMFULL = _PERM64 + [CD + _p for _p in _PERM64]

_mesh = plsc.VectorSubcoreMesh(
    core_axis_name="c", subcore_axis_name="s", num_cores=NC, num_subcores=NS
)
_sc_params = pltpu.CompilerParams(
    needs_layout_passes=False, use_tc_tiling_on_sc=False
)


def _deg_body(srcp, dstp, degs_out, degd_out, sidx, didx, degs, degd):
    c = lax.axis_index("c")
    s = lax.axis_index("s")
    wid = c * NS + s
    pltpu.sync_copy(srcp.at[wid], sidx)
    pltpu.sync_copy(dstp.at[wid], didx)
    zeros = jnp.zeros((L,), jnp.float32)

    def zero_body(i, _):
        degs[pl.ds(i * L, L)] = zeros
        degd[pl.ds(i * L, L)] = zeros
        return 0

    lax.fori_loop(0, NP // L, zero_body, 0)
    ones = jnp.ones((L,), jnp.float32)

    def cnt(j, _):
        for g in range(EB // L):
            plsc.addupdate_scatter(degs, [sidx[j, pl.ds(g * L, L)]], ones)
            plsc.addupdate_scatter(degd, [didx[j, pl.ds(g * L, L)]], ones)
        return 0

    lax.fori_loop(0, NB, cnt, 0)
    pltpu.sync_copy(degs, degs_out.at[wid])
    pltpu.sync_copy(degd, degd_out.at[wid])


_deg_call = pl.kernel(
    _deg_body,
    out_type=(
        jax.ShapeDtypeStruct((NW, NP), jnp.float32),
        jax.ShapeDtypeStruct((NW, NP), jnp.float32),
    ),
    mesh=_mesh,
    scratch_types=[
        pltpu.VMEM((NB, EB), jnp.int32),
        pltpu.VMEM((NB, EB), jnp.int32),
        pltpu.VMEM((NP,), jnp.float32),
        pltpu.VMEM((NP,), jnp.float32),
    ],
    compiler_params=_sc_params,
)


def _msg_body(table, srcp, dstp, ewp, parts,
              sidx, didx, ewv, b0, b1, b2, f0, f1, f2, agg,
              gsem0, gsem1, gsem2, ssem0, ssem1, ssem2):
    c = lax.axis_index("c")
    s = lax.axis_index("s")
    pltpu.sync_copy(srcp.at[s], sidx)
    pltpu.sync_copy(dstp.at[s], didx)
    pltpu.sync_copy(ewp.at[s], ewv)

    coff = c * NP

    def off(j, _):
        for g in range(EB // L):
            sl = pl.ds(g * L, L)
            sidx[j, sl] = sidx[j, sl] + coff
        return 0

    lax.fori_loop(0, NB3, off, 0)

    zeros = jnp.zeros((L,), jnp.float32)

    def zb(i, _):
        f0[i // (CD // L), pl.ds((i % (CD // L)) * L, L)] = zeros
        return 0

    lax.fori_loop(0, EB * (CD // L), zb, 0)

    def zs(k, _):
        pltpu.sync_copy(f0, agg.at[pl.ds(s * RPT + k * EB, EB)])
        return 0

    lax.fori_loop(0, RPT // EB, zs, 0)
    plsc.subcore_barrier()

    bbufs = (b0, b1, b2)
    gsems = (gsem0, gsem1, gsem2)
    fbufs = (f0, f1, f2)
    ssems = (ssem0, ssem1, ssem2)

    def gather(i, buf, sem):
        return pltpu.async_copy(table.at[sidx.at[i]], buf, sem)

    def gather_wait(i, buf, sem):
        pltpu.make_async_copy(table.at[sidx.at[i]], buf, sem).wait()

    def scatter(i, buf, sem):
        return pltpu.async_copy(buf, agg.at[didx.at[i]], sem, add=True)

    def scatter_wait(i, buf, sem):
        pltpu.make_async_copy(buf, agg.at[didx.at[i]], sem).wait()

    def scale(i, bsrc, fdst):
        def sc_body(g2, _):
            wv = ewv[i, pl.ds(g2 * 32, 32)]
            wa, wb = plsc.unpack(wv, format=plsc.PackFormat.INTERLEAVED)
            for k in range(L):
                for par, wvec in ((0, wa), (1, wb)):
                    w = wvec[k]
                    e = g2 * 32 + 2 * k + par
                    for g in range(CD // 32):
                        v = bsrc[e, pl.ds(g * 32, 32)]
                        a, b = plsc.unpack(
                            v, format=plsc.PackFormat.INTERLEAVED)
                        fdst[e, pl.ds(g * 32, L)] = a * w
                        fdst[e, pl.ds(g * 32 + L, L)] = b * w
            return 0

        lax.fori_loop(0, EB // 32, sc_body, 0)

    gather(0, b0, gsem0)
    gather(1, b1, gsem1)

    def blk3(j3, _):
        for t in range(3):
            j = j3 * 3 + t
            gather_wait(j, bbufs[t], gsems[t])

            @pl.when(j3 >= 1)
            def _():
                scatter_wait(j - 3, fbufs[t], ssems[t])

            scale(j, bbufs[t], fbufs[t])
            pre = (t + 2) % 3
            if t == 0:
                gather(j + 2, bbufs[pre], gsems[pre])
            else:
                @pl.when(j3 < NB3 // 3 - 1)
                def _():
                    gather(j + 2, bbufs[pre], gsems[pre])

            scatter(j, fbufs[t], ssems[t])
        return 0

    lax.fori_loop(0, NB3 // 3, blk3, 0)
    for t in range(3):
        scatter_wait(NB3 - 3 + t, fbufs[t], ssems[t])
    plsc.subcore_barrier()
    pltpu.sync_copy(agg.at[pl.ds(s * RPT, RPT)], parts.at[c, pl.ds(s * RPT, RPT)])


_msg_call = pl.kernel(
    _msg_body,
    out_type=jax.ShapeDtypeStruct((NC, NP, CD), jnp.float32),
    mesh=_mesh,
    scratch_types=[
        pltpu.VMEM((NB3, EB), jnp.int32),
        pltpu.VMEM((NB3, EB), jnp.int32),
        pltpu.VMEM((NB3, EB), jnp.bfloat16),
        pltpu.VMEM((EB, CD), jnp.bfloat16),
        pltpu.VMEM((EB, CD), jnp.bfloat16),
        pltpu.VMEM((EB, CD), jnp.bfloat16),
        pltpu.VMEM((EB, CD), jnp.float32),
        pltpu.VMEM((EB, CD), jnp.float32),
        pltpu.VMEM((EB, CD), jnp.float32),
        pltpu.VMEM_SHARED((NP, CD), jnp.float32),
        pltpu.SemaphoreType.DMA,
        pltpu.SemaphoreType.DMA,
        pltpu.SemaphoreType.DMA,
        pltpu.SemaphoreType.DMA,
        pltpu.SemaphoreType.DMA,
        pltpu.SemaphoreType.DMA,
    ],
    compiler_params=_sc_params,
)


def _norm_body(ds_ref, dd_ref, ns_ref, nd_ref):
    degs = jnp.sum(ds_ref[...], axis=0)
    degd = jnp.sum(dd_ref[...], axis=0)
    ns_ref[...] = lax.rsqrt(jnp.maximum(degs, 1.0))
    nd_ref[...] = lax.rsqrt(jnp.maximum(degd, 1.0))


_norm_call = pl.pallas_call(
    _norm_body,
    out_shape=(
        jax.ShapeDtypeStruct((NP // 128, 128), jnp.float32),
        jax.ShapeDtypeStruct((NP // 128, 128), jnp.float32),
    ),
)


def _scalet_body(xs_ref, ns_ref, out_ref):
    out_ref[...] = (xs_ref[...] * ns_ref[...]).astype(jnp.bfloat16)


_scale_table = pl.pallas_call(
    _scalet_body,
    grid=(NP // 512,),
    in_specs=[
        pl.BlockSpec((NC, 512, CD), lambda i: (0, i, 0)),
        pl.BlockSpec((512, 1), lambda i: (i, 0)),
    ],
    out_specs=pl.BlockSpec((NC, 512, CD), lambda i: (0, i, 0)),
    out_shape=jax.ShapeDtypeStruct((NC, NP, CD), jnp.bfloat16),
)


def _make_dense(with_res, ns_scale):
    def body(*refs):
        refs = list(refs)
        parts_ref = refs.pop(0)
        nd_ref = refs.pop(0)
        ns_ref = refs.pop(0) if ns_scale else None
        w_ref, b_ref, g_ref, be_ref = refs[:4]
        refs = refs[4:]
        res_ref = refs.pop(0) if with_res else None
        out_ref = refs.pop(0)
        p = jnp.concatenate([parts_ref[0], parts_ref[1]], axis=1) * nd_ref[...]
        acc = jnp.dot(p, w_ref[...], preferred_element_type=jnp.float32)
        y = g_ref[...] * ((acc + b_ref[...]) * BNS) + be_ref[...]
        if with_res:
            y = y + jnp.concatenate([res_ref[0], res_ref[1]], axis=1)
        if ns_scale:
            y = y * ns_ref[...]
            out_ref[0] = y[:, :CD].astype(jnp.bfloat16)
            out_ref[1] = y[:, CD:].astype(jnp.bfloat16)
        else:
            out_ref[0] = y[:, :CD]
            out_ref[1] = y[:, CD:]

    R = 512
    in_specs = [
        pl.BlockSpec((NC, R, CD), lambda i: (0, i, 0)),
        pl.BlockSpec((R, 1), lambda i: (i, 0)),
    ]
    if ns_scale:
        in_specs.append(pl.BlockSpec((R, 1), lambda i: (i, 0)))
    in_specs += [
        pl.BlockSpec((D, D), lambda i: (0, 0)),
        pl.BlockSpec((1, D), lambda i: (0, 0)),
        pl.BlockSpec((1, D), lambda i: (0, 0)),
        pl.BlockSpec((1, D), lambda i: (0, 0)),
    ]
    if with_res:
        in_specs.append(pl.BlockSpec((NC, R, CD), lambda i: (0, i, 0)))
    out_dtype = jnp.bfloat16 if ns_scale else jnp.float32
    return pl.pallas_call(
        body,
        grid=(NP // R,),
        in_specs=in_specs,
        out_specs=pl.BlockSpec((NC, R, CD), lambda i: (0, i, 0)),
        out_shape=jax.ShapeDtypeStruct((NC, NP, CD), out_dtype),
    )


_dense_mid = _make_dense(False, True)
_dense_final = _make_dense(True, False)


def kernel(x, edge_index, edge_weight, W1, b1, g1, be1, W2, b2, g2, be2):
    src = edge_index[0]
    dst = edge_index[1]
    pad = EP - E
    pad2 = EP2 - E
    srcp32 = jnp.concatenate(
        [src, jnp.full((pad,), N, jnp.int32)]).reshape(NW, NB, EB)
    dstp32 = jnp.concatenate(
        [dst, jnp.full((pad,), N, jnp.int32)]).reshape(NW, NB, EB)
    srcp16 = jnp.concatenate(
        [src, jnp.full((pad2,), N, jnp.int32)]).reshape(NS, NB3, EB)
    dstp16 = jnp.concatenate(
        [dst, jnp.full((pad2,), N, jnp.int32)]).reshape(NS, NB3, EB)
    ewp16 = jnp.concatenate(
        [edge_weight, jnp.zeros((pad2,), jnp.float32)]
    ).astype(jnp.bfloat16).reshape(NS, NB3, EB)

    xp = jnp.zeros((NP, D), jnp.float32).at[:N].set(x)
    xs = jnp.stack([xp[:, :CD], xp[:, CD:]])

    degs_p, degd_p = _deg_call(srcp32, dstp32)
    ns80, nd80 = _norm_call(
        degs_p.reshape(NW, NP // 128, 128), degd_p.reshape(NW, NP // 128, 128)
    )
    nscol = ns80.reshape(NP, 1)
    ndcol = nd80.reshape(NP, 1)

    perm = jnp.array(_PERMFULL, dtype=jnp.int32)
    W1p = W1[perm, :]
    W2p = W2[perm, :]
    b1r, g1r, be1r = b1.reshape(1, D), g1.reshape(1, D), be1.reshape(1, D)
    b2r, g2r, be2r = b2.reshape(1, D), g2.reshape(1, D), be2.reshape(1, D)

    xss = _scale_table(xs, nscol)
    parts1 = _msg_call(xss.reshape(NC * NP, CD), srcp16, dstp16, ewp16)
    h1s = _dense_mid(parts1, ndcol, nscol, W1p, b1r, g1r, be1r)
    parts2 = _msg_call(h1s.reshape(NC * NP, CD), srcp16, dstp16, ewp16)
    out = _dense_final(parts2, ndcol, W2p, b2r, g2r, be2r, xs)
    return jnp.concatenate([out[0, :N], out[1, :N]], axis=1)
